# baseline (speedup 1.0000x reference)
import functools

import jax
import jax.numpy as jnp
from jax.experimental import pallas as pl

D = 128


def _sage_block(mean_ref, x_ref, wl_ref, wr_ref, b_ref, o_ref):
    acc = jnp.dot(mean_ref[...], wl_ref[...], preferred_element_type=jnp.float32)
    acc += jnp.dot(x_ref[...], wr_ref[...], preferred_element_type=jnp.float32)
    o_ref[...] = jnp.maximum(acc + b_ref[...], 0.0)


def _sage_dense(mean, x_dst, Wl, Wr, b, block=1000):
    n = x_dst.shape[0]
    grid = (n // block,)
    return pl.pallas_call(
        _sage_block,
        grid=grid,
        in_specs=[
            pl.BlockSpec((block, D), lambda i: (i, 0)),
            pl.BlockSpec((block, D), lambda i: (i, 0)),
            pl.BlockSpec((D, D), lambda i: (0, 0)),
            pl.BlockSpec((D, D), lambda i: (0, 0)),
            pl.BlockSpec((1, D), lambda i: (0, 0)),
        ],
        out_specs=pl.BlockSpec((block, D), lambda i: (i, 0)),
        out_shape=jax.ShapeDtypeStruct((n, D), jnp.float32),
    )(mean, x_dst, Wl, Wr, b.reshape(1, D))


def _lin_block(x_ref, w_ref, b_ref, o_ref):
    o_ref[...] = (
        jnp.dot(x_ref[...], w_ref[...], preferred_element_type=jnp.float32)
        + b_ref[...]
    )


def _linear(x, W, b, block=1000):
    n = x.shape[0]
    return pl.pallas_call(
        _lin_block,
        grid=(n // block,),
        in_specs=[
            pl.BlockSpec((block, D), lambda i: (i, 0)),
            pl.BlockSpec((D, D), lambda i: (0, 0)),
            pl.BlockSpec((1, D), lambda i: (0, 0)),
        ],
        out_specs=pl.BlockSpec((block, D), lambda i: (i, 0)),
        out_shape=jax.ShapeDtypeStruct((n, D), jnp.float32),
    )(x, W, b.reshape(1, D))


def _dec_block(zs_ref, zt_ref, w1a_ref, w1b_ref, b1_ref, w2_ref, b2_ref, o_ref):
    h = jnp.dot(zs_ref[...], w1a_ref[...], preferred_element_type=jnp.float32)
    h += jnp.dot(zt_ref[...], w1b_ref[...], preferred_element_type=jnp.float32)
    h = jnp.maximum(h + b1_ref[...], 0.0)
    o_ref[...] = jnp.sum(h * w2_ref[...], axis=1, keepdims=True) + b2_ref[...]


def _decoder(zs_g, zt_g, W1, b1, W2, b2, block=2000):
    n = zs_g.shape[0]
    return pl.pallas_call(
        _dec_block,
        grid=(n // block,),
        in_specs=[
            pl.BlockSpec((block, D), lambda i: (i, 0)),
            pl.BlockSpec((block, D), lambda i: (i, 0)),
            pl.BlockSpec((D, D), lambda i: (0, 0)),
            pl.BlockSpec((D, D), lambda i: (0, 0)),
            pl.BlockSpec((1, D), lambda i: (0, 0)),
            pl.BlockSpec((1, D), lambda i: (0, 0)),
            pl.BlockSpec((1, 1), lambda i: (0, 0)),
        ],
        out_specs=pl.BlockSpec((block, 1), lambda i: (i, 0)),
        out_shape=jax.ShapeDtypeStruct((n, 1), jnp.float32),
    )(zs_g, zt_g, W1[:D], W1[D:], b1.reshape(1, D), W2.reshape(1, D), b2.reshape(1, 1))


def _seg_mean(x_src, ei, n_dst, inv_cnt):
    agg = jax.ops.segment_sum(x_src[ei[0]], ei[1], num_segments=n_dst)
    return agg * inv_cnt


def kernel(x_sotu, x_taxon, edge_index_st, edge_index_ts, edge_label_index,
           Wl1_st, bl1_st, Wr1_st, Wl1_ts, bl1_ts, Wr1_ts,
           Wl2_st, bl2_st, Wr2_st, Wl2_ts, bl2_ts, Wr2_ts,
           Wlin_s, blin_s, Wlin_t, blin_t, W1, b1, W2, b2):
    n_s, n_t = x_sotu.shape[0], x_taxon.shape[0]
    ones_st = jnp.ones((edge_index_st.shape[1],), jnp.float32)
    ones_ts = jnp.ones((edge_index_ts.shape[1],), jnp.float32)
    cnt_t = jax.ops.segment_sum(ones_st, edge_index_st[1], num_segments=n_t)
    cnt_s = jax.ops.segment_sum(ones_ts, edge_index_ts[1], num_segments=n_s)
    inv_t = (1.0 / jnp.maximum(cnt_t, 1.0))[:, None]
    inv_s = (1.0 / jnp.maximum(cnt_s, 1.0))[:, None]

    m1_t = _seg_mean(x_sotu, edge_index_st, n_t, inv_t)
    m1_s = _seg_mean(x_taxon, edge_index_ts, n_s, inv_s)
    h1_t = _sage_dense(m1_t, x_taxon, Wl1_st, Wr1_st, bl1_st)
    h1_s = _sage_dense(m1_s, x_sotu, Wl1_ts, Wr1_ts, bl1_ts)

    m2_t = _seg_mean(h1_s, edge_index_st, n_t, inv_t)
    m2_s = _seg_mean(h1_t, edge_index_ts, n_s, inv_s)
    h2_t = _sage_dense(m2_t, h1_t, Wl2_st, Wr2_st, bl2_st)
    h2_s = _sage_dense(m2_s, h1_s, Wl2_ts, Wr2_ts, bl2_ts)

    z_s = _linear(h2_s, Wlin_s, blin_s)
    z_t = _linear(h2_t, Wlin_t, blin_t)

    zs_g = z_s[edge_label_index[0]]
    zt_g = z_t[edge_label_index[1]]
    out = _decoder(zs_g, zt_g, W1, b1, W2, b2)
    return out.reshape(-1)
